# baseline (speedup 1.0000x reference)
import functools

import jax
import jax.numpy as jnp
from jax.experimental import pallas as pl
from jax.experimental.pallas import tpu as pltpu


def _cleanup_pieces(bm, nb):
    chunks = [(c * 1024, 1024) for c in range(9)] + [(9216, 768)]
    pieces = []
    for c, (col0, w) in enumerate(chunks):
        end = col0 + w
        t_c = min(nb, -(-end // bm))
        total_rows = t_c * bm
        r0 = 0
        while r0 < total_rows:
            nr = min(800, total_rows - r0)
            pieces.append((r0, nr, col0, w))
            r0 += nr
    return pieces


def _gcn_body(x_ref, a_ref, w1_ref, b1_ref, w2_ref, b2_ref, out_ref,
              sw_ref, s2_ref, strip_ref, abuf_ref, cbuf_ref, sems, csems,
              *, bm, nb, nbuf, cbuf_n, pieces):
    n = sw_ref.shape[0]
    nhid = w1_ref.shape[1]
    nout = w2_ref.shape[1]

    def row_desc(t, slot):
        return pltpu.make_async_copy(
            a_ref.at[pl.ds(t * bm, bm), :], abuf_ref.at[slot], sems.at[slot])

    def issue_row(t):
        @pl.when(t < nb)
        def _():
            row_desc(t, jax.lax.rem(t, nbuf)).start()

    def piece_desc(k, slot):
        r0, nr, c0, w = pieces[k]
        return pltpu.make_async_copy(
            a_ref.at[pl.ds(r0, nr), pl.ds(c0, w)],
            cbuf_ref.at[slot, pl.ds(0, nr), pl.ds(0, w)],
            csems.at[slot])

    for f0 in range(nbuf):
        issue_row(jnp.int32(f0))

    sw_ref[:, :nhid] = jnp.dot(x_ref[...], w1_ref[...],
                               preferred_element_type=jnp.float32)
    sw_ref[:, nhid:] = jnp.zeros_like(sw_ref[:, nhid:])

    def p1_step(t, carry):
        slot = jax.lax.rem(t, nbuf)
        row_desc(t, slot).wait()
        ablk = abuf_ref[slot]
        m = jnp.dot(ablk, sw_ref[...], preferred_element_type=jnp.float32)
        h = jnp.maximum(m[:, :nhid] + b1_ref[...], 0.0)
        s2_ref[pl.ds(t * bm, bm), :] = jnp.dot(
            h, w2_ref[...], preferred_element_type=jnp.float32)
        out_ref[pl.ds(t * bm, bm), :] = m[:, nhid:] + b2_ref[...]
        strip_ref[pl.ds(t * bm, bm), :] = abuf_ref[slot, :, pl.ds(9984, 16)]

        fc_old = (t * bm) // 1024
        fc_new = ((t + 1) * bm) // 1024

        @pl.when(fc_new > fc_old)
        def _():
            sw_ref[pl.ds(fc_old * 1024, 1024), nhid:] = (
                s2_ref[pl.ds(fc_old * 1024, 1024), :])

        @pl.when(t == nb - 1 - cbuf_n)
        def _():
            for k0 in range(min(cbuf_n, len(pieces))):
                piece_desc(k0, k0).start()

        issue_row(t + nbuf)
        return carry

    jax.lax.fori_loop(0, nb, p1_step, 0)

    for k, (r0, nr, c0, w) in enumerate(pieces):
        slot = k % cbuf_n
        piece_desc(k, slot).wait()
        out_ref[pl.ds(r0, nr), :] += jnp.dot(
            cbuf_ref[slot, pl.ds(0, nr), pl.ds(0, w)],
            s2_ref[pl.ds(c0, w), :], preferred_element_type=jnp.float32)
        if k + cbuf_n < len(pieces):
            piece_desc(k + cbuf_n, slot).start()

    out_ref[...] += jnp.dot(strip_ref[...], s2_ref[pl.ds(9984, 16), :],
                            preferred_element_type=jnp.float32)

    lsb = 2000
    nlsb = n // lsb

    def max_step(i, mx):
        return jnp.maximum(
            mx, jnp.max(out_ref[pl.ds(i * lsb, lsb), :], axis=0,
                        keepdims=True))

    mx = jax.lax.fori_loop(
        0, nlsb, max_step,
        jnp.full((1, nout), -jnp.inf, dtype=jnp.float32))

    def sum_step(i, s):
        return s + jnp.sum(jnp.exp(out_ref[pl.ds(i * lsb, lsb), :] - mx),
                           axis=0, keepdims=True)

    s = jax.lax.fori_loop(
        0, nlsb, sum_step, jnp.zeros((1, nout), dtype=jnp.float32))
    lse = jnp.log(s) + mx

    def sub_step(i, carry):
        out_ref[pl.ds(i * lsb, lsb), :] = (
            out_ref[pl.ds(i * lsb, lsb), :] - lse)
        return carry

    jax.lax.fori_loop(0, nlsb, sub_step, 0)


def kernel(features, adj_matrix, W1, b1, W2, b2):
    n, nin = features.shape
    nhid = W1.shape[1]
    nout = W2.shape[1]
    bm = 80
    nb = n // bm
    nbuf = 7
    cbuf_n = 3
    pieces = _cleanup_pieces(bm, nb)
    b1r = b1.reshape(1, nhid)
    b2r = b2.reshape(1, nout)

    body = functools.partial(_gcn_body, bm=bm, nb=nb, nbuf=nbuf,
                             cbuf_n=cbuf_n, pieces=pieces)
    out = pl.pallas_call(
        body,
        in_specs=[
            pl.BlockSpec(memory_space=pltpu.MemorySpace.VMEM),
            pl.BlockSpec(memory_space=pl.ANY),
            pl.BlockSpec(memory_space=pltpu.MemorySpace.VMEM),
            pl.BlockSpec(memory_space=pltpu.MemorySpace.VMEM),
            pl.BlockSpec(memory_space=pltpu.MemorySpace.VMEM),
            pl.BlockSpec(memory_space=pltpu.MemorySpace.VMEM),
        ],
        out_specs=pl.BlockSpec(memory_space=pltpu.MemorySpace.VMEM),
        out_shape=jax.ShapeDtypeStruct((n, nout), jnp.float32),
        scratch_shapes=[
            pltpu.VMEM((n, nhid + nout), jnp.float32),
            pltpu.VMEM((n, nout), jnp.float32),
            pltpu.VMEM((n, 16), jnp.float32),
            pltpu.VMEM((nbuf, bm, n), jnp.float32),
            pltpu.VMEM((cbuf_n, 800, 1024), jnp.float32),
            pltpu.SemaphoreType.DMA((nbuf,)),
            pltpu.SemaphoreType.DMA((cbuf_n,)),
        ],
    )(features, adj_matrix, W1, b1r, W2, b2r)
    return out


# speedup vs baseline: 1.0296x; 1.0296x over previous
import functools

import jax
import jax.numpy as jnp
from jax.experimental import pallas as pl
from jax.experimental.pallas import tpu as pltpu


def _cleanup_pieces(bm, nb):
    chunks = [(c * 1024, 1024) for c in range(9)] + [(9216, 768)]
    pieces = []
    for c, (col0, w) in enumerate(chunks):
        end = col0 + w
        t_c = min(nb, -(-end // bm))
        total_rows = t_c * bm
        r0 = 0
        while r0 < total_rows:
            nr = min(800, total_rows - r0)
            pieces.append((r0, nr, col0, w))
            r0 += nr
    return pieces


def _gcn_body(x_ref, a_ref, w1_ref, b1_ref, w2_ref, b2_ref, out_ref,
              sw_ref, s2_ref, strip_ref, abuf_ref, cbuf_ref, sems, csems,
              *, bm, nb, nbuf, cbuf_n, pieces):
    n = sw_ref.shape[0]
    nhid = w1_ref.shape[1]
    nout = w2_ref.shape[1]

    def row_desc(t, slot):
        return pltpu.make_async_copy(
            a_ref.at[pl.ds(t * bm, bm), :], abuf_ref.at[slot], sems.at[slot])

    def issue_row(t):
        @pl.when(t < nb)
        def _():
            row_desc(t, jax.lax.rem(t, nbuf)).start()

    def piece_desc(k, slot):
        r0, nr, c0, w = pieces[k]
        return pltpu.make_async_copy(
            a_ref.at[pl.ds(r0, nr), pl.ds(c0, w)],
            cbuf_ref.at[slot, pl.ds(0, nr), pl.ds(0, w)],
            csems.at[slot])

    for f0 in range(nbuf):
        issue_row(jnp.int32(f0))

    sw_ref[:, :nhid] = jnp.dot(x_ref[...], w1_ref[...],
                               preferred_element_type=jnp.float32)
    sw_ref[:, nhid:] = jnp.zeros_like(sw_ref[:, nhid:])

    def p1_step(t, carry):
        slot = jax.lax.rem(t, nbuf)
        row_desc(t, slot).wait()
        ablk = abuf_ref[slot]
        m = jnp.dot(ablk, sw_ref[...], preferred_element_type=jnp.float32)
        h = jnp.maximum(m[:, :nhid] + b1_ref[...], 0.0)
        s2_ref[pl.ds(t * bm, bm), :] = jnp.dot(
            h, w2_ref[...], preferred_element_type=jnp.float32)
        out_ref[pl.ds(t * bm, bm), :] = m[:, nhid:] + b2_ref[...]
        strip_ref[pl.ds(t * bm, bm), :] = abuf_ref[slot, :, pl.ds(9984, 16)]

        fc_old = (t * bm) // 1024
        fc_new = ((t + 1) * bm) // 1024

        @pl.when(fc_new > fc_old)
        def _():
            sw_ref[pl.ds(fc_old * 1024, 1024), nhid:] = (
                s2_ref[pl.ds(fc_old * 1024, 1024), :])

        @pl.when(t == nb - 1 - cbuf_n)
        def _():
            for k0 in range(min(cbuf_n, len(pieces))):
                piece_desc(k0, k0).start()

        issue_row(t + nbuf)
        return carry

    jax.lax.fori_loop(0, nb, p1_step, 0)

    for k, (r0, nr, c0, w) in enumerate(pieces):
        slot = k % cbuf_n
        piece_desc(k, slot).wait()
        out_ref[pl.ds(r0, nr), :] += jnp.dot(
            cbuf_ref[slot, pl.ds(0, nr), pl.ds(0, w)],
            s2_ref[pl.ds(c0, w), :], preferred_element_type=jnp.float32)
        if k + cbuf_n < len(pieces):
            piece_desc(k + cbuf_n, slot).start()

    out_ref[...] += jnp.dot(strip_ref[...], s2_ref[pl.ds(9984, 16), :],
                            preferred_element_type=jnp.float32)

    lsb = 2000
    nlsb = n // lsb

    def max_step(i, mx):
        return jnp.maximum(
            mx, jnp.max(out_ref[pl.ds(i * lsb, lsb), :], axis=0,
                        keepdims=True))

    mx = jax.lax.fori_loop(
        0, nlsb, max_step,
        jnp.full((1, nout), -jnp.inf, dtype=jnp.float32))

    def sum_step(i, s):
        return s + jnp.sum(jnp.exp(out_ref[pl.ds(i * lsb, lsb), :] - mx),
                           axis=0, keepdims=True)

    s = jax.lax.fori_loop(
        0, nlsb, sum_step, jnp.zeros((1, nout), dtype=jnp.float32))
    lse = jnp.log(s) + mx

    def sub_step(i, carry):
        out_ref[pl.ds(i * lsb, lsb), :] = (
            out_ref[pl.ds(i * lsb, lsb), :] - lse)
        return carry

    jax.lax.fori_loop(0, nlsb, sub_step, 0)


def kernel(features, adj_matrix, W1, b1, W2, b2):
    n, nin = features.shape
    nhid = W1.shape[1]
    nout = W2.shape[1]
    bm = 80
    nb = n // bm
    nbuf = 6
    cbuf_n = 4
    pieces = _cleanup_pieces(bm, nb)
    b1r = b1.reshape(1, nhid)
    b2r = b2.reshape(1, nout)

    body = functools.partial(_gcn_body, bm=bm, nb=nb, nbuf=nbuf,
                             cbuf_n=cbuf_n, pieces=pieces)
    out = pl.pallas_call(
        body,
        in_specs=[
            pl.BlockSpec(memory_space=pltpu.MemorySpace.VMEM),
            pl.BlockSpec(memory_space=pl.ANY),
            pl.BlockSpec(memory_space=pltpu.MemorySpace.VMEM),
            pl.BlockSpec(memory_space=pltpu.MemorySpace.VMEM),
            pl.BlockSpec(memory_space=pltpu.MemorySpace.VMEM),
            pl.BlockSpec(memory_space=pltpu.MemorySpace.VMEM),
        ],
        out_specs=pl.BlockSpec(memory_space=pltpu.MemorySpace.VMEM),
        out_shape=jax.ShapeDtypeStruct((n, nout), jnp.float32),
        scratch_shapes=[
            pltpu.VMEM((n, nhid + nout), jnp.float32),
            pltpu.VMEM((n, nout), jnp.float32),
            pltpu.VMEM((n, 16), jnp.float32),
            pltpu.VMEM((nbuf, bm, n), jnp.float32),
            pltpu.VMEM((cbuf_n, 800, 1024), jnp.float32),
            pltpu.SemaphoreType.DMA((nbuf,)),
            pltpu.SemaphoreType.DMA((cbuf_n,)),
        ],
    )(features, adj_matrix, W1, b1r, W2, b2r)
    return out
